# Initial kernel scaffold; baseline (speedup 1.0000x reference)
import jax
import jax.numpy as jnp
from jax.experimental import pallas as pl


def kernel(atomic_numbers, positions, edge_index, emb, centers, widths, Wrbf, Wj, bj, Wi, bi, Wri1, bri1, Wri2, bri2, Wra1, bra1, Wra2, bra2, u):
    raise NotImplementedError("write your pallas kernel here")


# speedup vs baseline: 3.0314x; 3.0314x over previous
import functools

import jax
import jax.numpy as jnp
from jax import lax
from jax.experimental import pallas as pl
from jax.experimental.pallas import tpu as pltpu
from jax.experimental.pallas import tpu_sc as plsc

N = 10000
E = 320000
F = 128
K = 128
RC = 10.0

NC, NS, L = 2, 16, 16
NW = NC * NS
NPAD = 10240
EPAD = 327680
NODES_PT = NPAD // NW
EDGES_PT = E // NW
CH = 128
NCH = (EPAD // NW) // CH
ZROWS = NPAD // NS
EB = 4096
NEB = EPAD // EB
NBLK = 256
LOG2 = 0.6931471805599453

def _act(x):
    return jnp.maximum(x, 0.0) + jnp.log(1.0 + jnp.exp(-jnp.abs(x))) - LOG2


def _sc_prep_body(z_hbm, emb_hbm, pos_hbm, src_hbm, dst_hbm,
                  x0_hbm, d2_hbm,
                  zidx, erows, posv, sbuf, dbuf, d2buf, sem):
    cid = lax.axis_index("c")
    sid = lax.axis_index("s")
    wid = sid * NC + cid
    nb = wid * NODES_PT
    for c in range(4):
        base = nb + c * 80
        pltpu.sync_copy(z_hbm.at[pl.ds(base, 80)], zidx)
        pltpu.async_copy(emb_hbm.at[zidx], erows, sem).wait()
        pltpu.sync_copy(erows, x0_hbm.at[pl.ds(base, 80)])
    eb = wid * EDGES_PT
    pltpu.sync_copy(pos_hbm, posv)
    pltpu.sync_copy(src_hbm.at[pl.ds(eb, EDGES_PT)], sbuf)
    pltpu.sync_copy(dst_hbm.at[pl.ds(eb, EDGES_PT)], dbuf)

    def body(i, carry):
        sl = pl.ds(i * L, L)
        sv = sbuf[sl] * 4
        dv = dbuf[sl] * 4
        xs = plsc.load_gather(posv, [sv])
        ys = plsc.load_gather(posv, [sv + 1])
        zs = plsc.load_gather(posv, [sv + 2])
        xd = plsc.load_gather(posv, [dv])
        yd = plsc.load_gather(posv, [dv + 1])
        zd = plsc.load_gather(posv, [dv + 2])
        dx = xd - xs
        dy = yd - ys
        dz = zd - zs
        d2buf[sl] = dx * dx + dy * dy + dz * dz
        return carry

    lax.fori_loop(0, EDGES_PT // L, body, 0)
    pltpu.sync_copy(d2buf, d2_hbm.at[pl.ds(eb, EDGES_PT)])


@functools.cache
def _sc_prep():
    mesh = plsc.VectorSubcoreMesh(core_axis_name="c", subcore_axis_name="s",
                                  num_cores=NC, num_subcores=NS)
    return pl.kernel(
        _sc_prep_body,
        out_type=[jax.ShapeDtypeStruct((NPAD, F), jnp.float32),
                  jax.ShapeDtypeStruct((E,), jnp.float32)],
        mesh=mesh,
        scratch_types=[
            pltpu.VMEM((80,), jnp.int32),
            pltpu.VMEM((80, F), jnp.float32),
            pltpu.VMEM((NPAD * 4,), jnp.float32),
            pltpu.VMEM((EDGES_PT,), jnp.int32),
            pltpu.VMEM((EDGES_PT,), jnp.int32),
            pltpu.VMEM((EDGES_PT,), jnp.float32),
            pltpu.SemaphoreType.DMA,
        ],
        compiler_params=pltpu.CompilerParams(needs_layout_passes=False),
    )


def _sc_msg_body(gate_hbm, hj_hbm, src_hbm, dst_hbm, zer_hbm,
                 agg_hbm,
                 idxs, idxd, gbuf, hbuf, shared, sem):
    cid = lax.axis_index("c")
    sid = lax.axis_index("s")
    wid = sid * NC + cid
    pltpu.sync_copy(zer_hbm, shared.at[pl.ds(sid * ZROWS, ZROWS)])
    plsc.subcore_barrier()
    eb = wid * (EPAD // NW)

    def chunk(i, carry):
        base = eb + i * CH
        pltpu.sync_copy(src_hbm.at[pl.ds(base, CH)], idxs)
        pltpu.sync_copy(dst_hbm.at[pl.ds(base, CH)], idxd)
        cp = pltpu.async_copy(hj_hbm.at[idxs], hbuf, sem)
        pltpu.sync_copy(gate_hbm.at[pl.ds(base, CH)], gbuf)
        cp.wait()

        def mulrow(r, c2):
            for c in range(F // L):
                sl = pl.ds(c * L, L)
                gbuf[r, sl] = gbuf[r, sl] * hbuf[r, sl]
            return c2

        lax.fori_loop(0, CH, mulrow, 0)
        pltpu.sync_copy(gbuf, shared.at[idxd], add=True)
        return carry

    lax.fori_loop(0, NCH, chunk, 0)
    plsc.subcore_barrier()
    orow = cid * NPAD + sid * ZROWS
    pltpu.sync_copy(shared.at[pl.ds(sid * ZROWS, ZROWS)],
                    agg_hbm.at[pl.ds(orow, ZROWS)])


@functools.cache
def _sc_msg():
    mesh = plsc.VectorSubcoreMesh(core_axis_name="c", subcore_axis_name="s",
                                  num_cores=NC, num_subcores=NS)
    return pl.kernel(
        _sc_msg_body,
        out_type=jax.ShapeDtypeStruct((NC * NPAD, F), jnp.float32),
        mesh=mesh,
        scratch_types=[
            pltpu.VMEM((CH,), jnp.int32),
            pltpu.VMEM((CH,), jnp.int32),
            pltpu.VMEM((CH, F), jnp.float32),
            pltpu.VMEM((CH, F), jnp.float32),
            pltpu.VMEM_SHARED((NPAD, F), jnp.float32),
            pltpu.SemaphoreType.DMA,
        ],
        compiler_params=pltpu.CompilerParams(needs_layout_passes=False),
    )


def _tc_gate_body(d2_ref, cen_ref, wid_ref, w0_ref, w1_ref, g0_ref, g1_ref):
    cen = cen_ref[...]
    wdt = wid_ref[...]
    d2b = d2_ref[...].reshape(EB // 512, 512)
    for r in range(EB // 512):
        d2r = d2b[r:r + 1, :]
        dij = jnp.sqrt(d2r + 1e-12)
        xr = dij * (1.0 / RC)
        phi = 1.0 + xr * xr * xr * (-10.0 + xr * (15.0 - 6.0 * xr))
        cf = jnp.where(dij < RC, phi, 0.0)
        ed = jnp.exp(-dij)
        z = ed - cen
        rbf_t = cf * jnp.exp(-wdt * z * z)
        dn = (((0,), (0,)), ((), ()))
        sl = pl.ds(r * 512, 512)
        g0_ref[sl, :] = lax.dot_general(rbf_t, w0_ref[...], dn,
                                        preferred_element_type=jnp.float32)
        g1_ref[sl, :] = lax.dot_general(rbf_t, w1_ref[...], dn,
                                        preferred_element_type=jnp.float32)


_tc_gates = pl.pallas_call(
    _tc_gate_body,
    grid=(NEB,),
    in_specs=[
        pl.BlockSpec((1, 8, 512), lambda i: (i, 0, 0)),
        pl.BlockSpec((K, 1), lambda i: (0, 0)),
        pl.BlockSpec((K, 1), lambda i: (0, 0)),
        pl.BlockSpec((K, F), lambda i: (0, 0)),
        pl.BlockSpec((K, F), lambda i: (0, 0)),
    ],
    out_specs=[
        pl.BlockSpec((EB, F), lambda i: (i, 0)),
        pl.BlockSpec((EB, F), lambda i: (i, 0)),
    ],
    out_shape=[jax.ShapeDtypeStruct((EPAD, F), jnp.float32),
               jax.ShapeDtypeStruct((EPAD, F), jnp.float32)],
)


def _tc_node_body(x_ref, wj_ref, bj_ref, wi_ref, bi_ref, hj_ref, hi_ref):
    xt = _act(x_ref[...])
    hj_ref[...] = _act(jnp.dot(xt, wj_ref[...],
                               preferred_element_type=jnp.float32) + bj_ref[...])
    hi_ref[...] = _act(jnp.dot(xt, wi_ref[...],
                               preferred_element_type=jnp.float32) + bi_ref[...])


_tc_node = pl.pallas_call(
    _tc_node_body,
    grid=(NPAD // NBLK,),
    in_specs=[
        pl.BlockSpec((NBLK, F), lambda i: (i, 0)),
        pl.BlockSpec((F, F), lambda i: (0, 0)),
        pl.BlockSpec((1, F), lambda i: (0, 0)),
        pl.BlockSpec((F, F), lambda i: (0, 0)),
        pl.BlockSpec((1, F), lambda i: (0, 0)),
    ],
    out_specs=[
        pl.BlockSpec((NBLK, F), lambda i: (i, 0)),
        pl.BlockSpec((NBLK, F), lambda i: (i, 0)),
    ],
    out_shape=[jax.ShapeDtypeStruct((NPAD, F), jnp.float32),
               jax.ShapeDtypeStruct((NPAD, F), jnp.float32)],
)


def _tc_update_body(x_ref, hi_ref, agg_ref, wri1_ref, bri1_ref, wri2_ref,
                    bri2_ref, wra1_ref, bra1_ref, wra2_ref, bra2_ref, u_ref,
                    xo_ref):
    agg = agg_ref[0] + agg_ref[1]
    v = hi_ref[...] + agg
    t = _act(jnp.dot(_act(v), wri1_ref[...],
                     preferred_element_type=jnp.float32) + bri1_ref[...])
    v = v + jnp.dot(t, wri2_ref[...],
                    preferred_element_type=jnp.float32) + bri2_ref[...]
    xn = u_ref[...] * x_ref[...] + v
    s = _act(jnp.dot(_act(xn), wra1_ref[...],
                     preferred_element_type=jnp.float32) + bra1_ref[...])
    xo_ref[...] = xn + jnp.dot(s, wra2_ref[...],
                               preferred_element_type=jnp.float32) + bra2_ref[...]


_tc_update = pl.pallas_call(
    _tc_update_body,
    grid=(NPAD // NBLK,),
    in_specs=[
        pl.BlockSpec((NBLK, F), lambda i: (i, 0)),
        pl.BlockSpec((NBLK, F), lambda i: (i, 0)),
        pl.BlockSpec((NC, NBLK, F), lambda i: (0, i, 0)),
        pl.BlockSpec((F, F), lambda i: (0, 0)),
        pl.BlockSpec((1, F), lambda i: (0, 0)),
        pl.BlockSpec((F, F), lambda i: (0, 0)),
        pl.BlockSpec((1, F), lambda i: (0, 0)),
        pl.BlockSpec((F, F), lambda i: (0, 0)),
        pl.BlockSpec((1, F), lambda i: (0, 0)),
        pl.BlockSpec((F, F), lambda i: (0, 0)),
        pl.BlockSpec((1, F), lambda i: (0, 0)),
        pl.BlockSpec((1, F), lambda i: (0, 0)),
    ],
    out_specs=pl.BlockSpec((NBLK, F), lambda i: (i, 0)),
    out_shape=jax.ShapeDtypeStruct((NPAD, F), jnp.float32),
)


def kernel(atomic_numbers, positions, edge_index, emb, centers, widths, Wrbf,
           Wj, bj, Wi, bi, Wri1, bri1, Wri2, bri2, Wra1, bra1, Wra2, bra2, u):
    f32 = jnp.float32
    src = edge_index[0].astype(jnp.int32)
    dst = edge_index[1].astype(jnp.int32)
    zpad = jnp.pad(atomic_numbers.astype(jnp.int32), (0, NPAD - N))
    posp = jnp.pad(positions.astype(f32),
                   ((0, NPAD - N), (0, 1))).reshape(NPAD * 4)
    srcp = jnp.pad(src, (0, EPAD - E))
    dstp = jnp.pad(dst, (0, EPAD - E))

    x0, d2 = _sc_prep()(zpad, emb, posp, src, dst)
    d2p = jnp.concatenate(
        [d2, jnp.full((EPAD - E,), 4.0 * RC * RC, f32)]).reshape(NEB, 8, 512)
    gate0, gate1 = _tc_gates(d2p, centers.reshape(K, 1), widths.reshape(K, 1),
                             Wrbf[0], Wrbf[1])
    zer = jnp.zeros((ZROWS, F), f32)

    x = x0
    gates = (gate0, gate1)
    for b in range(2):
        hj, hi = _tc_node(x, Wj[b], bj[b].reshape(1, F), Wi[b],
                          bi[b].reshape(1, F))
        agg = _sc_msg()(gates[b], hj, srcp, dstp, zer).reshape(NC, NPAD, F)
        x = _tc_update(x, hi, agg, Wri1[b], bri1[b].reshape(1, F), Wri2[b],
                       bri2[b].reshape(1, F), Wra1[b], bra1[b].reshape(1, F),
                       Wra2[b], bra2[b].reshape(1, F), u[b].reshape(1, F))
    return x[:N]


# speedup vs baseline: 3.3715x; 1.1122x over previous
import functools

import jax
import jax.numpy as jnp
from jax import lax
from jax.experimental import pallas as pl
from jax.experimental.pallas import tpu as pltpu
from jax.experimental.pallas import tpu_sc as plsc

N = 10000
E = 320000
F = 128
K = 128
RC = 10.0

NC, NS, L = 2, 16, 16
NW = NC * NS
NPAD = 10240
EPAD = 327680
NODES_PT = NPAD // NW
EDGES_PT = E // NW
CH = 80
NCH = (EPAD // NW) // CH
ZROWS = NPAD // NS
EB = 4096
NEB = EPAD // EB
NBLK = 256
LOG2 = 0.6931471805599453

def _act(x):
    return jnp.maximum(x, 0.0) + jnp.log(1.0 + jnp.exp(-jnp.abs(x))) - LOG2


def _sc_prep_body(z_hbm, emb_hbm, pos_hbm, src_hbm, dst_hbm,
                  x0_hbm, d2_hbm,
                  zidx, erows, posv, sbuf, dbuf, d2buf, sem):
    cid = lax.axis_index("c")
    sid = lax.axis_index("s")
    wid = sid * NC + cid
    nb = wid * NODES_PT
    for c in range(4):
        base = nb + c * 80
        pltpu.sync_copy(z_hbm.at[pl.ds(base, 80)], zidx)
        pltpu.async_copy(emb_hbm.at[zidx], erows, sem).wait()
        pltpu.sync_copy(erows, x0_hbm.at[pl.ds(base, 80)])
    eb = wid * EDGES_PT
    pltpu.sync_copy(pos_hbm, posv)
    pltpu.sync_copy(src_hbm.at[pl.ds(eb, EDGES_PT)], sbuf)
    pltpu.sync_copy(dst_hbm.at[pl.ds(eb, EDGES_PT)], dbuf)

    def body(i, carry):
        sl = pl.ds(i * L, L)
        sv = sbuf[sl] * 4
        dv = dbuf[sl] * 4
        xs = plsc.load_gather(posv, [sv])
        ys = plsc.load_gather(posv, [sv + 1])
        zs = plsc.load_gather(posv, [sv + 2])
        xd = plsc.load_gather(posv, [dv])
        yd = plsc.load_gather(posv, [dv + 1])
        zd = plsc.load_gather(posv, [dv + 2])
        dx = xd - xs
        dy = yd - ys
        dz = zd - zs
        d2buf[sl] = dx * dx + dy * dy + dz * dz
        return carry

    lax.fori_loop(0, EDGES_PT // L, body, 0)
    pltpu.sync_copy(d2buf, d2_hbm.at[pl.ds(eb, EDGES_PT)])


@functools.cache
def _sc_prep():
    mesh = plsc.VectorSubcoreMesh(core_axis_name="c", subcore_axis_name="s",
                                  num_cores=NC, num_subcores=NS)
    return pl.kernel(
        _sc_prep_body,
        out_type=[jax.ShapeDtypeStruct((NPAD, F), jnp.float32),
                  jax.ShapeDtypeStruct((E,), jnp.float32)],
        mesh=mesh,
        scratch_types=[
            pltpu.VMEM((80,), jnp.int32),
            pltpu.VMEM((80, F), jnp.float32),
            pltpu.VMEM((NPAD * 4,), jnp.float32),
            pltpu.VMEM((EDGES_PT,), jnp.int32),
            pltpu.VMEM((EDGES_PT,), jnp.int32),
            pltpu.VMEM((EDGES_PT,), jnp.float32),
            pltpu.SemaphoreType.DMA,
        ],
        compiler_params=pltpu.CompilerParams(needs_layout_passes=False),
    )


def _sc_msg_body(gate_hbm, hj_hbm, src_hbm, dst_hbm, zer_hbm,
                 agg_hbm,
                 ixs0, ixd0, ixs1, ixd1, sd0, sd1, gbuf0, gbuf1, hbuf0, hbuf1,
                 shared, is0, is1, gs0, gs1, ts0, ts1, ss0, ss1):
    cid = lax.axis_index("c")
    sid = lax.axis_index("s")
    wid = sid * NC + cid
    eb = wid * (EPAD // NW)
    ixs = (ixs0, ixs1)
    ixd = (ixd0, ixd1)
    sctd = (sd0, sd1)
    gbuf = (gbuf0, gbuf1)
    hbuf = (hbuf0, hbuf1)
    isem = (is0, is1)
    gsem = (gs0, gs1)
    tsem = (ts0, ts1)
    ssem = (ss0, ss1)

    def start_idx(c, s):
        pltpu.async_copy(src_hbm.at[pl.ds(eb + c * CH, CH)], ixs[s], isem[s])
        pltpu.async_copy(dst_hbm.at[pl.ds(eb + c * CH, CH)], ixd[s], isem[s])

    def wait_idx(c, s):
        pltpu.make_async_copy(
            src_hbm.at[pl.ds(eb + c * CH, CH)], ixs[s], isem[s]).wait()
        pltpu.make_async_copy(
            dst_hbm.at[pl.ds(eb + c * CH, CH)], ixd[s], isem[s]).wait()

    def start_gather(s):
        pltpu.async_copy(hj_hbm.at[ixs[s]], hbuf[s], gsem[s])

    def start_gate(c, s):
        pltpu.async_copy(gate_hbm.at[pl.ds(eb + c * CH, CH)], gbuf[s], tsem[s])

    def wait_in(c, s):
        pltpu.make_async_copy(hj_hbm.at[ixs[s]], hbuf[s], gsem[s]).wait()
        pltpu.make_async_copy(
            gate_hbm.at[pl.ds(eb + c * CH, CH)], gbuf[s], tsem[s]).wait()

    def mul(s):
        gb, hb = gbuf[s], hbuf[s]

        @plsc.parallel_loop(0, CH, step=1, unroll=2)
        def _(r):
            for c in range(F // L):
                sl = pl.ds(c * L, L)
                gb[r, sl] = gb[r, sl] * hb[r, sl]

    def stage_sidx(s):
        for k in range(CH // L):
            sl = pl.ds(k * L, L)
            sctd[s][sl] = ixd[s][sl]

    def scat_start(s):
        pltpu.async_copy(gbuf[s], shared.at[sctd[s]], ssem[s], add=True)

    def scat_wait(s):
        pltpu.make_async_copy(gbuf[s], shared.at[sctd[s]], ssem[s]).wait()

    start_idx(0, 0)
    start_idx(1, 1)
    wait_idx(0, 0)
    start_gather(0)
    start_gate(0, 0)
    pltpu.sync_copy(zer_hbm, shared.at[pl.ds(sid * ZROWS, ZROWS)])
    plsc.subcore_barrier()

    def pair(g, carry):
        for sub in (0, 1):
            c = 2 * g + sub
            s = sub
            o = 1 - sub
            wait_in(c, s)
            mul(s)

            @pl.when(c + 1 < NCH)
            def _():
                wait_idx(c + 1, o)
                start_gather(o)

            @pl.when((c > 0) & (c + 1 < NCH))
            def _():
                scat_wait(o)

            @pl.when(c + 1 < NCH)
            def _():
                start_gate(c + 1, o)

            stage_sidx(s)

            @pl.when(c + 2 < NCH)
            def _():
                start_idx(c + 2, s)

            scat_start(s)
        return carry

    lax.fori_loop(0, NCH // 2, pair, 0)
    scat_wait(0)
    scat_wait(1)
    plsc.subcore_barrier()
    orow = cid * NPAD + sid * ZROWS
    pltpu.sync_copy(shared.at[pl.ds(sid * ZROWS, ZROWS)],
                    agg_hbm.at[pl.ds(orow, ZROWS)])


@functools.cache
def _sc_msg():
    mesh = plsc.VectorSubcoreMesh(core_axis_name="c", subcore_axis_name="s",
                                  num_cores=NC, num_subcores=NS)
    return pl.kernel(
        _sc_msg_body,
        out_type=jax.ShapeDtypeStruct((NC * NPAD, F), jnp.float32),
        mesh=mesh,
        scratch_types=[
            pltpu.VMEM((CH,), jnp.int32),
            pltpu.VMEM((CH,), jnp.int32),
            pltpu.VMEM((CH,), jnp.int32),
            pltpu.VMEM((CH,), jnp.int32),
            pltpu.VMEM((CH,), jnp.int32),
            pltpu.VMEM((CH,), jnp.int32),
            pltpu.VMEM((CH, F), jnp.float32),
            pltpu.VMEM((CH, F), jnp.float32),
            pltpu.VMEM((CH, F), jnp.float32),
            pltpu.VMEM((CH, F), jnp.float32),
            pltpu.VMEM_SHARED((NPAD, F), jnp.float32),
            pltpu.SemaphoreType.DMA,
            pltpu.SemaphoreType.DMA,
            pltpu.SemaphoreType.DMA,
            pltpu.SemaphoreType.DMA,
            pltpu.SemaphoreType.DMA,
            pltpu.SemaphoreType.DMA,
            pltpu.SemaphoreType.DMA,
            pltpu.SemaphoreType.DMA,
        ],
        compiler_params=pltpu.CompilerParams(needs_layout_passes=False),
    )


def _tc_gate_body(d2_ref, cen_ref, wid_ref, w0_ref, w1_ref, g0_ref, g1_ref):
    cen = cen_ref[...]
    wdt = wid_ref[...]
    d2b = d2_ref[...].reshape(EB // 512, 512)
    for r in range(EB // 512):
        d2r = d2b[r:r + 1, :]
        dij = jnp.sqrt(d2r + 1e-12)
        xr = dij * (1.0 / RC)
        phi = 1.0 + xr * xr * xr * (-10.0 + xr * (15.0 - 6.0 * xr))
        cf = jnp.where(dij < RC, phi, 0.0)
        ed = jnp.exp(-dij)
        z = ed - cen
        rbf_t = cf * jnp.exp(-wdt * z * z)
        dn = (((0,), (0,)), ((), ()))
        sl = pl.ds(r * 512, 512)
        g0_ref[sl, :] = lax.dot_general(rbf_t, w0_ref[...], dn,
                                        preferred_element_type=jnp.float32)
        g1_ref[sl, :] = lax.dot_general(rbf_t, w1_ref[...], dn,
                                        preferred_element_type=jnp.float32)


_tc_gates = pl.pallas_call(
    _tc_gate_body,
    grid=(NEB,),
    in_specs=[
        pl.BlockSpec((1, 8, 512), lambda i: (i, 0, 0)),
        pl.BlockSpec((K, 1), lambda i: (0, 0)),
        pl.BlockSpec((K, 1), lambda i: (0, 0)),
        pl.BlockSpec((K, F), lambda i: (0, 0)),
        pl.BlockSpec((K, F), lambda i: (0, 0)),
    ],
    out_specs=[
        pl.BlockSpec((EB, F), lambda i: (i, 0)),
        pl.BlockSpec((EB, F), lambda i: (i, 0)),
    ],
    out_shape=[jax.ShapeDtypeStruct((EPAD, F), jnp.float32),
               jax.ShapeDtypeStruct((EPAD, F), jnp.float32)],
)


def _tc_node_body(x_ref, wj_ref, bj_ref, wi_ref, bi_ref, hj_ref, hi_ref):
    xt = _act(x_ref[...])
    hj_ref[...] = _act(jnp.dot(xt, wj_ref[...],
                               preferred_element_type=jnp.float32) + bj_ref[...])
    hi_ref[...] = _act(jnp.dot(xt, wi_ref[...],
                               preferred_element_type=jnp.float32) + bi_ref[...])


_tc_node = pl.pallas_call(
    _tc_node_body,
    grid=(NPAD // NBLK,),
    in_specs=[
        pl.BlockSpec((NBLK, F), lambda i: (i, 0)),
        pl.BlockSpec((F, F), lambda i: (0, 0)),
        pl.BlockSpec((1, F), lambda i: (0, 0)),
        pl.BlockSpec((F, F), lambda i: (0, 0)),
        pl.BlockSpec((1, F), lambda i: (0, 0)),
    ],
    out_specs=[
        pl.BlockSpec((NBLK, F), lambda i: (i, 0)),
        pl.BlockSpec((NBLK, F), lambda i: (i, 0)),
    ],
    out_shape=[jax.ShapeDtypeStruct((NPAD, F), jnp.float32),
               jax.ShapeDtypeStruct((NPAD, F), jnp.float32)],
)


def _tc_update_body(x_ref, hi_ref, agg_ref, wri1_ref, bri1_ref, wri2_ref,
                    bri2_ref, wra1_ref, bra1_ref, wra2_ref, bra2_ref, u_ref,
                    xo_ref):
    agg = agg_ref[0] + agg_ref[1]
    v = hi_ref[...] + agg
    t = _act(jnp.dot(_act(v), wri1_ref[...],
                     preferred_element_type=jnp.float32) + bri1_ref[...])
    v = v + jnp.dot(t, wri2_ref[...],
                    preferred_element_type=jnp.float32) + bri2_ref[...]
    xn = u_ref[...] * x_ref[...] + v
    s = _act(jnp.dot(_act(xn), wra1_ref[...],
                     preferred_element_type=jnp.float32) + bra1_ref[...])
    xo_ref[...] = xn + jnp.dot(s, wra2_ref[...],
                               preferred_element_type=jnp.float32) + bra2_ref[...]


_tc_update = pl.pallas_call(
    _tc_update_body,
    grid=(NPAD // NBLK,),
    in_specs=[
        pl.BlockSpec((NBLK, F), lambda i: (i, 0)),
        pl.BlockSpec((NBLK, F), lambda i: (i, 0)),
        pl.BlockSpec((NC, NBLK, F), lambda i: (0, i, 0)),
        pl.BlockSpec((F, F), lambda i: (0, 0)),
        pl.BlockSpec((1, F), lambda i: (0, 0)),
        pl.BlockSpec((F, F), lambda i: (0, 0)),
        pl.BlockSpec((1, F), lambda i: (0, 0)),
        pl.BlockSpec((F, F), lambda i: (0, 0)),
        pl.BlockSpec((1, F), lambda i: (0, 0)),
        pl.BlockSpec((F, F), lambda i: (0, 0)),
        pl.BlockSpec((1, F), lambda i: (0, 0)),
        pl.BlockSpec((1, F), lambda i: (0, 0)),
    ],
    out_specs=pl.BlockSpec((NBLK, F), lambda i: (i, 0)),
    out_shape=jax.ShapeDtypeStruct((NPAD, F), jnp.float32),
)


def kernel(atomic_numbers, positions, edge_index, emb, centers, widths, Wrbf,
           Wj, bj, Wi, bi, Wri1, bri1, Wri2, bri2, Wra1, bra1, Wra2, bra2, u):
    f32 = jnp.float32
    src = edge_index[0].astype(jnp.int32)
    dst = edge_index[1].astype(jnp.int32)
    zpad = jnp.pad(atomic_numbers.astype(jnp.int32), (0, NPAD - N))
    posp = jnp.pad(positions.astype(f32),
                   ((0, NPAD - N), (0, 1))).reshape(NPAD * 4)
    srcp = jnp.pad(src, (0, EPAD - E))
    dstp = jnp.pad(dst, (0, EPAD - E))

    x0, d2 = _sc_prep()(zpad, emb, posp, src, dst)
    d2p = jnp.concatenate(
        [d2, jnp.full((EPAD - E,), 4.0 * RC * RC, f32)]).reshape(NEB, 8, 512)
    gate0, gate1 = _tc_gates(d2p, centers.reshape(K, 1), widths.reshape(K, 1),
                             Wrbf[0], Wrbf[1])
    zer = jnp.zeros((ZROWS, F), f32)

    x = x0
    gates = (gate0, gate1)
    for b in range(2):
        hj, hi = _tc_node(x, Wj[b], bj[b].reshape(1, F), Wi[b],
                          bi[b].reshape(1, F))
        agg = _sc_msg()(gates[b], hj, srcp, dstp, zer).reshape(NC, NPAD, F)
        x = _tc_update(x, hi, agg, Wri1[b], bri1[b].reshape(1, F), Wri2[b],
                       bri2[b].reshape(1, F), Wra1[b], bra1[b].reshape(1, F),
                       Wra2[b], bra2[b].reshape(1, F), u[b].reshape(1, F))
    return x[:N]


# speedup vs baseline: 3.6174x; 1.0729x over previous
import functools

import jax
import jax.numpy as jnp
from jax import lax
from jax.experimental import pallas as pl
from jax.experimental.pallas import tpu as pltpu
from jax.experimental.pallas import tpu_sc as plsc

N = 10000
E = 320000
F = 128
K = 128
RC = 10.0

NC, NS, L = 2, 16, 16
NW = NC * NS
NPAD = 10240
EPAD = 327680
NODES_PT = NPAD // NW
EDGES_PT = E // NW
CH = 80
NCH = (EPAD // NW) // CH
ZROWS = NPAD // NS
EB = 4096
NEB = EPAD // EB
NBLK = 256
LOG2 = 0.6931471805599453

def _act(x):
    return jnp.maximum(x, 0.0) + jnp.log(1.0 + jnp.exp(-jnp.abs(x))) - LOG2


def _sc_prep_body(z_hbm, emb_hbm, pos_hbm, src_hbm, dst_hbm,
                  x0_hbm, d2_hbm,
                  zidx, erows, posv, sbuf, dbuf, d2buf, sem):
    cid = lax.axis_index("c")
    sid = lax.axis_index("s")
    wid = sid * NC + cid
    nb = wid * NODES_PT
    for c in range(4):
        base = nb + c * 80
        pltpu.sync_copy(z_hbm.at[pl.ds(base, 80)], zidx)
        pltpu.async_copy(emb_hbm.at[zidx], erows, sem).wait()
        pltpu.sync_copy(erows, x0_hbm.at[pl.ds(base, 80)])
    eb = wid * EDGES_PT
    pltpu.sync_copy(pos_hbm, posv)
    pltpu.sync_copy(src_hbm.at[pl.ds(eb, EDGES_PT)], sbuf)
    pltpu.sync_copy(dst_hbm.at[pl.ds(eb, EDGES_PT)], dbuf)

    def body(i, carry):
        sl = pl.ds(i * L, L)
        sv = sbuf[sl] * 4
        dv = dbuf[sl] * 4
        xs = plsc.load_gather(posv, [sv])
        ys = plsc.load_gather(posv, [sv + 1])
        zs = plsc.load_gather(posv, [sv + 2])
        xd = plsc.load_gather(posv, [dv])
        yd = plsc.load_gather(posv, [dv + 1])
        zd = plsc.load_gather(posv, [dv + 2])
        dx = xd - xs
        dy = yd - ys
        dz = zd - zs
        d2buf[sl] = dx * dx + dy * dy + dz * dz
        return carry

    lax.fori_loop(0, EDGES_PT // L, body, 0)
    pltpu.sync_copy(d2buf, d2_hbm.at[pl.ds(eb, EDGES_PT)])


@functools.cache
def _sc_prep():
    mesh = plsc.VectorSubcoreMesh(core_axis_name="c", subcore_axis_name="s",
                                  num_cores=NC, num_subcores=NS)
    return pl.kernel(
        _sc_prep_body,
        out_type=[jax.ShapeDtypeStruct((NPAD, F), jnp.float32),
                  jax.ShapeDtypeStruct((E,), jnp.float32)],
        mesh=mesh,
        scratch_types=[
            pltpu.VMEM((80,), jnp.int32),
            pltpu.VMEM((80, F), jnp.float32),
            pltpu.VMEM((NPAD * 4,), jnp.float32),
            pltpu.VMEM((EDGES_PT,), jnp.int32),
            pltpu.VMEM((EDGES_PT,), jnp.int32),
            pltpu.VMEM((EDGES_PT,), jnp.float32),
            pltpu.SemaphoreType.DMA,
        ],
        compiler_params=pltpu.CompilerParams(needs_layout_passes=False),
    )


def _sc_msg_body(gate_hbm, hj_hbm, src_hbm, dst_hbm, zer_hbm,
                 agg_hbm,
                 ixs0, ixd0, ixs1, ixd1, sd0, sd1, gbuf0, gbuf1, hbuf0, hbuf1,
                 shared, is0, is1, gs0, gs1, ts0, ts1, ss0, ss1):
    cid = lax.axis_index("c")
    sid = lax.axis_index("s")
    wid = sid * NC + cid
    eb = wid * (EPAD // NW)
    ixs = (ixs0, ixs1)
    ixd = (ixd0, ixd1)
    sctd = (sd0, sd1)
    gbuf = (gbuf0, gbuf1)
    hbuf = (hbuf0, hbuf1)
    isem = (is0, is1)
    gsem = (gs0, gs1)
    tsem = (ts0, ts1)
    ssem = (ss0, ss1)

    def start_idx(c, s):
        pltpu.async_copy(src_hbm.at[pl.ds(eb + c * CH, CH)], ixs[s], isem[s])
        pltpu.async_copy(dst_hbm.at[pl.ds(eb + c * CH, CH)], ixd[s], isem[s])

    def wait_idx(c, s):
        pltpu.make_async_copy(
            src_hbm.at[pl.ds(eb + c * CH, CH)], ixs[s], isem[s]).wait()
        pltpu.make_async_copy(
            dst_hbm.at[pl.ds(eb + c * CH, CH)], ixd[s], isem[s]).wait()

    def start_gather(s):
        pltpu.async_copy(hj_hbm.at[ixs[s]], hbuf[s], gsem[s])

    def start_gate(c, s):
        base = pl.multiple_of((eb + c * CH) // 2, 8)
        pltpu.async_copy(gate_hbm.at[pl.ds(base, CH // 2)], gbuf[s], tsem[s])

    def wait_in(c, s):
        pltpu.make_async_copy(hj_hbm.at[ixs[s]], hbuf[s], gsem[s]).wait()
        base = pl.multiple_of((eb + c * CH) // 2, 8)
        pltpu.make_async_copy(
            gate_hbm.at[pl.ds(base, CH // 2)], gbuf[s], tsem[s]).wait()

    def mul(s):
        gb, hb = gbuf[s], hbuf[s]

        @plsc.parallel_loop(0, CH // 2, step=1, unroll=2)
        def _(r):
            for c in range(F // L):
                sl = pl.ds(L * c, L)
                gg = plsc.bitcast(gb[r, sl], jnp.bfloat16)
                ge, go = plsc.unpack(gg, format=plsc.PackFormat.INTERLEAVED)
                hb[2 * r, sl] = ge * hb[2 * r, sl]
                hb[2 * r + 1, sl] = go * hb[2 * r + 1, sl]

    def stage_sidx(s):
        for k in range(CH // L):
            sl = pl.ds(k * L, L)
            sctd[s][sl] = ixd[s][sl]

    def scat_start(s):
        pltpu.async_copy(hbuf[s], shared.at[sctd[s]], ssem[s], add=True)

    def scat_wait(s):
        pltpu.make_async_copy(hbuf[s], shared.at[sctd[s]], ssem[s]).wait()

    start_idx(0, 0)
    start_idx(1, 1)
    wait_idx(0, 0)
    start_gather(0)
    start_gate(0, 0)
    pltpu.sync_copy(zer_hbm, shared.at[pl.ds(sid * ZROWS, ZROWS)])
    plsc.subcore_barrier()

    def pair(g, carry):
        for sub in (0, 1):
            c = 2 * g + sub
            s = sub
            o = 1 - sub
            wait_in(c, s)
            mul(s)

            @pl.when((c > 0) & (c + 1 < NCH))
            def _():
                scat_wait(o)

            @pl.when(c + 1 < NCH)
            def _():
                wait_idx(c + 1, o)
                start_gather(o)
                start_gate(c + 1, o)

            stage_sidx(s)

            @pl.when(c + 2 < NCH)
            def _():
                start_idx(c + 2, s)

            scat_start(s)
        return carry

    lax.fori_loop(0, NCH // 2, pair, 0)
    scat_wait(0)
    scat_wait(1)
    plsc.subcore_barrier()
    orow = cid * NPAD + sid * ZROWS
    pltpu.sync_copy(shared.at[pl.ds(sid * ZROWS, ZROWS)],
                    agg_hbm.at[pl.ds(orow, ZROWS)])


@functools.cache
def _sc_msg():
    mesh = plsc.VectorSubcoreMesh(core_axis_name="c", subcore_axis_name="s",
                                  num_cores=NC, num_subcores=NS)
    return pl.kernel(
        _sc_msg_body,
        out_type=jax.ShapeDtypeStruct((NC * NPAD, F), jnp.float32),
        mesh=mesh,
        scratch_types=[
            pltpu.VMEM((CH,), jnp.int32),
            pltpu.VMEM((CH,), jnp.int32),
            pltpu.VMEM((CH,), jnp.int32),
            pltpu.VMEM((CH,), jnp.int32),
            pltpu.VMEM((CH,), jnp.int32),
            pltpu.VMEM((CH,), jnp.int32),
            pltpu.VMEM((CH // 2, F), jnp.int32),
            pltpu.VMEM((CH // 2, F), jnp.int32),
            pltpu.VMEM((CH, F), jnp.float32),
            pltpu.VMEM((CH, F), jnp.float32),
            pltpu.VMEM_SHARED((NPAD, F), jnp.float32),
            pltpu.SemaphoreType.DMA,
            pltpu.SemaphoreType.DMA,
            pltpu.SemaphoreType.DMA,
            pltpu.SemaphoreType.DMA,
            pltpu.SemaphoreType.DMA,
            pltpu.SemaphoreType.DMA,
            pltpu.SemaphoreType.DMA,
            pltpu.SemaphoreType.DMA,
        ],
        compiler_params=pltpu.CompilerParams(needs_layout_passes=False),
    )


def _tc_gate_body(d2_ref, cen_ref, wid_ref, w0_ref, w1_ref, g0_ref, g1_ref):
    cen = cen_ref[...]
    wdt = wid_ref[...]
    d2b = d2_ref[...].reshape(EB // 512, 512)
    for r in range(EB // 512):
        d2r = d2b[r:r + 1, :]
        dij = jnp.sqrt(d2r + 1e-12)
        xr = dij * (1.0 / RC)
        phi = 1.0 + xr * xr * xr * (-10.0 + xr * (15.0 - 6.0 * xr))
        cf = jnp.where(dij < RC, phi, 0.0)
        ed = jnp.exp(-dij)
        z = ed - cen
        rbf_t = cf * jnp.exp(-wdt * z * z)
        dn = (((0,), (0,)), ((), ()))
        sl = pl.ds(r * 256, 256)
        g0 = lax.dot_general(rbf_t, w0_ref[...], dn,
                             preferred_element_type=jnp.float32)
        g1 = lax.dot_general(rbf_t, w1_ref[...], dn,
                             preferred_element_type=jnp.float32)
        g0_ref[sl, :] = pltpu.bitcast(g0.astype(jnp.bfloat16), jnp.int32)
        g1_ref[sl, :] = pltpu.bitcast(g1.astype(jnp.bfloat16), jnp.int32)


_tc_gates = pl.pallas_call(
    _tc_gate_body,
    grid=(NEB,),
    in_specs=[
        pl.BlockSpec((1, 8, 512), lambda i: (i, 0, 0)),
        pl.BlockSpec((K, 1), lambda i: (0, 0)),
        pl.BlockSpec((K, 1), lambda i: (0, 0)),
        pl.BlockSpec((K, F), lambda i: (0, 0)),
        pl.BlockSpec((K, F), lambda i: (0, 0)),
    ],
    out_specs=[
        pl.BlockSpec((EB // 2, F), lambda i: (i, 0)),
        pl.BlockSpec((EB // 2, F), lambda i: (i, 0)),
    ],
    out_shape=[jax.ShapeDtypeStruct((EPAD // 2, F), jnp.int32),
               jax.ShapeDtypeStruct((EPAD // 2, F), jnp.int32)],
)


def _tc_node_body(x_ref, wj_ref, bj_ref, wi_ref, bi_ref, hj_ref, hi_ref):
    xt = _act(x_ref[...])
    hj_ref[...] = _act(jnp.dot(xt, wj_ref[...],
                               preferred_element_type=jnp.float32) + bj_ref[...])
    hi_ref[...] = _act(jnp.dot(xt, wi_ref[...],
                               preferred_element_type=jnp.float32) + bi_ref[...])


_tc_node = pl.pallas_call(
    _tc_node_body,
    grid=(NPAD // NBLK,),
    in_specs=[
        pl.BlockSpec((NBLK, F), lambda i: (i, 0)),
        pl.BlockSpec((F, F), lambda i: (0, 0)),
        pl.BlockSpec((1, F), lambda i: (0, 0)),
        pl.BlockSpec((F, F), lambda i: (0, 0)),
        pl.BlockSpec((1, F), lambda i: (0, 0)),
    ],
    out_specs=[
        pl.BlockSpec((NBLK, F), lambda i: (i, 0)),
        pl.BlockSpec((NBLK, F), lambda i: (i, 0)),
    ],
    out_shape=[jax.ShapeDtypeStruct((NPAD, F), jnp.float32),
               jax.ShapeDtypeStruct((NPAD, F), jnp.float32)],
)


def _tc_update_body(x_ref, hi_ref, agg_ref, wri1_ref, bri1_ref, wri2_ref,
                    bri2_ref, wra1_ref, bra1_ref, wra2_ref, bra2_ref, u_ref,
                    xo_ref):
    agg = agg_ref[0] + agg_ref[1]
    v = hi_ref[...] + agg
    t = _act(jnp.dot(_act(v), wri1_ref[...],
                     preferred_element_type=jnp.float32) + bri1_ref[...])
    v = v + jnp.dot(t, wri2_ref[...],
                    preferred_element_type=jnp.float32) + bri2_ref[...]
    xn = u_ref[...] * x_ref[...] + v
    s = _act(jnp.dot(_act(xn), wra1_ref[...],
                     preferred_element_type=jnp.float32) + bra1_ref[...])
    xo_ref[...] = xn + jnp.dot(s, wra2_ref[...],
                               preferred_element_type=jnp.float32) + bra2_ref[...]


_tc_update = pl.pallas_call(
    _tc_update_body,
    grid=(NPAD // NBLK,),
    in_specs=[
        pl.BlockSpec((NBLK, F), lambda i: (i, 0)),
        pl.BlockSpec((NBLK, F), lambda i: (i, 0)),
        pl.BlockSpec((NC, NBLK, F), lambda i: (0, i, 0)),
        pl.BlockSpec((F, F), lambda i: (0, 0)),
        pl.BlockSpec((1, F), lambda i: (0, 0)),
        pl.BlockSpec((F, F), lambda i: (0, 0)),
        pl.BlockSpec((1, F), lambda i: (0, 0)),
        pl.BlockSpec((F, F), lambda i: (0, 0)),
        pl.BlockSpec((1, F), lambda i: (0, 0)),
        pl.BlockSpec((F, F), lambda i: (0, 0)),
        pl.BlockSpec((1, F), lambda i: (0, 0)),
        pl.BlockSpec((1, F), lambda i: (0, 0)),
    ],
    out_specs=pl.BlockSpec((NBLK, F), lambda i: (i, 0)),
    out_shape=jax.ShapeDtypeStruct((NPAD, F), jnp.float32),
)


def kernel(atomic_numbers, positions, edge_index, emb, centers, widths, Wrbf,
           Wj, bj, Wi, bi, Wri1, bri1, Wri2, bri2, Wra1, bra1, Wra2, bra2, u):
    f32 = jnp.float32
    src = edge_index[0].astype(jnp.int32)
    dst = edge_index[1].astype(jnp.int32)
    zpad = jnp.pad(atomic_numbers.astype(jnp.int32), (0, NPAD - N))
    posp = jnp.pad(positions.astype(f32),
                   ((0, NPAD - N), (0, 1))).reshape(NPAD * 4)
    srcp = jnp.pad(src, (0, EPAD - E))
    dstp = jnp.pad(dst, (0, EPAD - E))

    x0, d2 = _sc_prep()(zpad, emb, posp, src, dst)
    d2p = jnp.concatenate(
        [d2, jnp.full((EPAD - E,), 4.0 * RC * RC, f32)]).reshape(NEB, 8, 512)
    gate0, gate1 = _tc_gates(d2p, centers.reshape(K, 1), widths.reshape(K, 1),
                             Wrbf[0], Wrbf[1])
    zer = jnp.zeros((ZROWS, F), f32)

    x = x0
    gates = (gate0, gate1)
    for b in range(2):
        hj, hi = _tc_node(x, Wj[b], bj[b].reshape(1, F), Wi[b],
                          bi[b].reshape(1, F))
        agg = _sc_msg()(gates[b], hj, srcp, dstp, zer).reshape(NC, NPAD, F)
        x = _tc_update(x, hi, agg, Wri1[b], bri1[b].reshape(1, F), Wri2[b],
                       bri2[b].reshape(1, F), Wra1[b], bra1[b].reshape(1, F),
                       Wra2[b], bra2[b].reshape(1, F), u[b].reshape(1, F))
    return x[:N]


# speedup vs baseline: 3.8904x; 1.0755x over previous
import functools

import jax
import jax.numpy as jnp
from jax import lax
from jax.experimental import pallas as pl
from jax.experimental.pallas import tpu as pltpu
from jax.experimental.pallas import tpu_sc as plsc

N = 10000
E = 320000
F = 128
K = 128
RC = 10.0

NC, NS, L = 2, 16, 16
NW = NC * NS
NPAD = 10240
EPAD = 327680
NODES_PT = NPAD // NW
EDGES_PT = E // NW
CH = 80
NCH = (EPAD // NW) // CH
NCH0 = 160
NCH1 = 2 * NCH - NCH0
ZROWS = NPAD // NS
EB = 4096
NEB = EPAD // EB
NBLK = 256
LOG2 = 0.6931471805599453

def _act(x):
    return jnp.maximum(x, 0.0) + jnp.log(1.0 + jnp.exp(-jnp.abs(x))) - LOG2


def _sc_prep_body(z_hbm, emb_hbm, pos_hbm, src_hbm, dst_hbm,
                  x0_hbm, d2_hbm,
                  zidx, erows, posv, sbuf, dbuf, d2buf, sem):
    cid = lax.axis_index("c")
    sid = lax.axis_index("s")
    wid = sid * NC + cid
    nb = wid * NODES_PT
    for c in range(4):
        base = nb + c * 80
        pltpu.sync_copy(z_hbm.at[pl.ds(base, 80)], zidx)
        pltpu.async_copy(emb_hbm.at[zidx], erows, sem).wait()
        pltpu.sync_copy(erows, x0_hbm.at[pl.ds(base, 80)])
    eb = wid * EDGES_PT
    pltpu.sync_copy(pos_hbm, posv)
    pltpu.sync_copy(src_hbm.at[pl.ds(eb, EDGES_PT)], sbuf)
    pltpu.sync_copy(dst_hbm.at[pl.ds(eb, EDGES_PT)], dbuf)

    def body(i, carry):
        sl = pl.ds(i * L, L)
        sv = sbuf[sl] * 4
        dv = dbuf[sl] * 4
        xs = plsc.load_gather(posv, [sv])
        ys = plsc.load_gather(posv, [sv + 1])
        zs = plsc.load_gather(posv, [sv + 2])
        xd = plsc.load_gather(posv, [dv])
        yd = plsc.load_gather(posv, [dv + 1])
        zd = plsc.load_gather(posv, [dv + 2])
        dx = xd - xs
        dy = yd - ys
        dz = zd - zs
        d2buf[sl] = dx * dx + dy * dy + dz * dz
        return carry

    lax.fori_loop(0, EDGES_PT // L, body, 0)
    pltpu.sync_copy(d2buf, d2_hbm.at[pl.ds(eb, EDGES_PT)])


@functools.cache
def _sc_prep():
    mesh = plsc.VectorSubcoreMesh(core_axis_name="c", subcore_axis_name="s",
                                  num_cores=NC, num_subcores=NS)
    return pl.kernel(
        _sc_prep_body,
        out_type=[jax.ShapeDtypeStruct((NPAD, F), jnp.float32),
                  jax.ShapeDtypeStruct((E,), jnp.float32)],
        mesh=mesh,
        scratch_types=[
            pltpu.VMEM((80,), jnp.int32),
            pltpu.VMEM((80, F), jnp.float32),
            pltpu.VMEM((NPAD * 4,), jnp.float32),
            pltpu.VMEM((EDGES_PT,), jnp.int32),
            pltpu.VMEM((EDGES_PT,), jnp.int32),
            pltpu.VMEM((EDGES_PT,), jnp.float32),
            pltpu.SemaphoreType.DMA,
        ],
        compiler_params=pltpu.CompilerParams(needs_layout_passes=False),
    )


def _sc_msg_body(gate_hbm, hj_hbm, src_hbm, dst_hbm, zer_hbm,
                 agg_hbm,
                 ixs0, ixd0, ixs1, ixd1, sd0, sd1, gbuf0, gbuf1, hbuf0, hbuf1,
                 shared, is0, is1, gs0, gs1, ts0, ts1, ss0, ss1):
    cid = lax.axis_index("c")
    sid = lax.axis_index("s")
    ixs = (ixs0, ixs1)
    ixd = (ixd0, ixd1)
    sctd = (sd0, sd1)
    gbuf = (gbuf0, gbuf1)
    hbuf = (hbuf0, hbuf1)
    isem = (is0, is1)
    gsem = (gs0, gs1)
    tsem = (ts0, ts1)
    ssem = (ss0, ss1)

    def start_idx(ch, s):
        eb = ch * CH
        pltpu.async_copy(src_hbm.at[pl.ds(eb, CH)], ixs[s], isem[s])
        pltpu.async_copy(dst_hbm.at[pl.ds(eb, CH)], ixd[s], isem[s])

    def wait_idx(ch, s):
        eb = ch * CH
        pltpu.make_async_copy(
            src_hbm.at[pl.ds(eb, CH)], ixs[s], isem[s]).wait()
        pltpu.make_async_copy(
            dst_hbm.at[pl.ds(eb, CH)], ixd[s], isem[s]).wait()

    def start_gather(s):
        pltpu.async_copy(hj_hbm.at[ixs[s]], hbuf[s], gsem[s])

    def start_gate(ch, s):
        base = pl.multiple_of(ch * (CH // 2), 8)
        pltpu.async_copy(gate_hbm.at[pl.ds(base, CH // 2)], gbuf[s], tsem[s])

    def wait_in(ch, s):
        pltpu.make_async_copy(hj_hbm.at[ixs[s]], hbuf[s], gsem[s]).wait()
        base = pl.multiple_of(ch * (CH // 2), 8)
        pltpu.make_async_copy(
            gate_hbm.at[pl.ds(base, CH // 2)], gbuf[s], tsem[s]).wait()

    def mul(s):
        gb, hb = gbuf[s], hbuf[s]

        @plsc.parallel_loop(0, CH // 2, step=1, unroll=2)
        def _(r):
            for c in range(F // L):
                sl = pl.ds(L * c, L)
                gg = plsc.bitcast(gb[r, sl], jnp.bfloat16)
                ge, go = plsc.unpack(gg, format=plsc.PackFormat.INTERLEAVED)
                hb[2 * r, sl] = ge * hb[2 * r, sl]
                hb[2 * r + 1, sl] = go * hb[2 * r + 1, sl]

    def stage_sidx(s):
        for k in range(CH // L):
            sl = pl.ds(k * L, L)
            sctd[s][sl] = ixd[s][sl]

    def scat_start(s):
        pltpu.async_copy(hbuf[s], shared.at[sctd[s]], ssem[s], add=True)

    def scat_wait(s):
        pltpu.make_async_copy(hbuf[s], shared.at[sctd[s]], ssem[s]).wait()

    def run(cbase, nch):
        start_idx(cbase, 0)
        start_idx(cbase + 1, 1)
        wait_idx(cbase, 0)
        start_gather(0)
        start_gate(cbase, 0)

        def pair(g, carry):
            for sub in (0, 1):
                c = 2 * g + sub
                s = sub
                o = 1 - sub
                wait_in(cbase + c, s)
                mul(s)

                @pl.when((c > 0) & (c + 1 < nch))
                def _():
                    scat_wait(o)

                @pl.when(c + 1 < nch)
                def _():
                    wait_idx(cbase + c + 1, o)
                    start_gather(o)
                    start_gate(cbase + c + 1, o)

                stage_sidx(s)

                @pl.when(c + 2 < nch)
                def _():
                    start_idx(cbase + c + 2, s)

                scat_start(s)
            return carry

        lax.fori_loop(0, nch // 2, pair, 0)
        scat_wait(0)
        scat_wait(1)

    pltpu.sync_copy(zer_hbm, shared.at[pl.ds(sid * ZROWS, ZROWS)])
    plsc.subcore_barrier()

    @pl.when(cid == 0)
    def _():
        run(sid * NCH0, NCH0)

    @pl.when(cid == 1)
    def _():
        run(NS * NCH0 + sid * NCH1, NCH1)

    plsc.subcore_barrier()
    orow = cid * NPAD + sid * ZROWS
    pltpu.sync_copy(shared.at[pl.ds(sid * ZROWS, ZROWS)],
                    agg_hbm.at[pl.ds(orow, ZROWS)])


@functools.cache
def _sc_msg():
    mesh = plsc.VectorSubcoreMesh(core_axis_name="c", subcore_axis_name="s",
                                  num_cores=NC, num_subcores=NS)
    return pl.kernel(
        _sc_msg_body,
        out_type=jax.ShapeDtypeStruct((NC * NPAD, F), jnp.float32),
        mesh=mesh,
        scratch_types=[
            pltpu.VMEM((CH,), jnp.int32),
            pltpu.VMEM((CH,), jnp.int32),
            pltpu.VMEM((CH,), jnp.int32),
            pltpu.VMEM((CH,), jnp.int32),
            pltpu.VMEM((CH,), jnp.int32),
            pltpu.VMEM((CH,), jnp.int32),
            pltpu.VMEM((CH // 2, F), jnp.int32),
            pltpu.VMEM((CH // 2, F), jnp.int32),
            pltpu.VMEM((CH, F), jnp.float32),
            pltpu.VMEM((CH, F), jnp.float32),
            pltpu.VMEM_SHARED((NPAD, F), jnp.float32),
            pltpu.SemaphoreType.DMA,
            pltpu.SemaphoreType.DMA,
            pltpu.SemaphoreType.DMA,
            pltpu.SemaphoreType.DMA,
            pltpu.SemaphoreType.DMA,
            pltpu.SemaphoreType.DMA,
            pltpu.SemaphoreType.DMA,
            pltpu.SemaphoreType.DMA,
        ],
        compiler_params=pltpu.CompilerParams(needs_layout_passes=False),
    )


def _tc_gate_body(d2_ref, cen_ref, wid_ref, w0_ref, w1_ref, g0_ref, g1_ref):
    cen = cen_ref[...]
    wdt = wid_ref[...]
    d2b = d2_ref[...].reshape(EB // 512, 512)
    for r in range(EB // 512):
        d2r = d2b[r:r + 1, :]
        dij = jnp.sqrt(d2r + 1e-12)
        xr = dij * (1.0 / RC)
        phi = 1.0 + xr * xr * xr * (-10.0 + xr * (15.0 - 6.0 * xr))
        cf = jnp.where(dij < RC, phi, 0.0)
        ed = jnp.exp(-dij)
        z = ed - cen
        rbf_t = cf * jnp.exp(-wdt * z * z)
        dn = (((0,), (0,)), ((), ()))
        sl = pl.ds(r * 256, 256)
        g0 = lax.dot_general(rbf_t, w0_ref[...], dn,
                             preferred_element_type=jnp.float32)
        g1 = lax.dot_general(rbf_t, w1_ref[...], dn,
                             preferred_element_type=jnp.float32)
        g0_ref[sl, :] = pltpu.bitcast(g0.astype(jnp.bfloat16), jnp.int32)
        g1_ref[sl, :] = pltpu.bitcast(g1.astype(jnp.bfloat16), jnp.int32)


_tc_gates = pl.pallas_call(
    _tc_gate_body,
    grid=(NEB,),
    in_specs=[
        pl.BlockSpec((1, 8, 512), lambda i: (i, 0, 0)),
        pl.BlockSpec((K, 1), lambda i: (0, 0)),
        pl.BlockSpec((K, 1), lambda i: (0, 0)),
        pl.BlockSpec((K, F), lambda i: (0, 0)),
        pl.BlockSpec((K, F), lambda i: (0, 0)),
    ],
    out_specs=[
        pl.BlockSpec((EB // 2, F), lambda i: (i, 0)),
        pl.BlockSpec((EB // 2, F), lambda i: (i, 0)),
    ],
    out_shape=[jax.ShapeDtypeStruct((EPAD // 2, F), jnp.int32),
               jax.ShapeDtypeStruct((EPAD // 2, F), jnp.int32)],
)


def _tc_node_body(x_ref, wj_ref, bj_ref, wi_ref, bi_ref, hj_ref, hi_ref):
    xt = _act(x_ref[...])
    hj_ref[...] = _act(jnp.dot(xt, wj_ref[...],
                               preferred_element_type=jnp.float32) + bj_ref[...])
    hi_ref[...] = _act(jnp.dot(xt, wi_ref[...],
                               preferred_element_type=jnp.float32) + bi_ref[...])


_tc_node = pl.pallas_call(
    _tc_node_body,
    grid=(NPAD // NBLK,),
    in_specs=[
        pl.BlockSpec((NBLK, F), lambda i: (i, 0)),
        pl.BlockSpec((F, F), lambda i: (0, 0)),
        pl.BlockSpec((1, F), lambda i: (0, 0)),
        pl.BlockSpec((F, F), lambda i: (0, 0)),
        pl.BlockSpec((1, F), lambda i: (0, 0)),
    ],
    out_specs=[
        pl.BlockSpec((NBLK, F), lambda i: (i, 0)),
        pl.BlockSpec((NBLK, F), lambda i: (i, 0)),
    ],
    out_shape=[jax.ShapeDtypeStruct((NPAD, F), jnp.float32),
               jax.ShapeDtypeStruct((NPAD, F), jnp.float32)],
)


def _tc_update_body(x_ref, hi_ref, agg_ref, wri1_ref, bri1_ref, wri2_ref,
                    bri2_ref, wra1_ref, bra1_ref, wra2_ref, bra2_ref, u_ref,
                    xo_ref):
    agg = agg_ref[0] + agg_ref[1]
    v = hi_ref[...] + agg
    t = _act(jnp.dot(_act(v), wri1_ref[...],
                     preferred_element_type=jnp.float32) + bri1_ref[...])
    v = v + jnp.dot(t, wri2_ref[...],
                    preferred_element_type=jnp.float32) + bri2_ref[...]
    xn = u_ref[...] * x_ref[...] + v
    s = _act(jnp.dot(_act(xn), wra1_ref[...],
                     preferred_element_type=jnp.float32) + bra1_ref[...])
    xo_ref[...] = xn + jnp.dot(s, wra2_ref[...],
                               preferred_element_type=jnp.float32) + bra2_ref[...]


_tc_update = pl.pallas_call(
    _tc_update_body,
    grid=(NPAD // NBLK,),
    in_specs=[
        pl.BlockSpec((NBLK, F), lambda i: (i, 0)),
        pl.BlockSpec((NBLK, F), lambda i: (i, 0)),
        pl.BlockSpec((NC, NBLK, F), lambda i: (0, i, 0)),
        pl.BlockSpec((F, F), lambda i: (0, 0)),
        pl.BlockSpec((1, F), lambda i: (0, 0)),
        pl.BlockSpec((F, F), lambda i: (0, 0)),
        pl.BlockSpec((1, F), lambda i: (0, 0)),
        pl.BlockSpec((F, F), lambda i: (0, 0)),
        pl.BlockSpec((1, F), lambda i: (0, 0)),
        pl.BlockSpec((F, F), lambda i: (0, 0)),
        pl.BlockSpec((1, F), lambda i: (0, 0)),
        pl.BlockSpec((1, F), lambda i: (0, 0)),
    ],
    out_specs=pl.BlockSpec((NBLK, F), lambda i: (i, 0)),
    out_shape=jax.ShapeDtypeStruct((NPAD, F), jnp.float32),
)


def kernel(atomic_numbers, positions, edge_index, emb, centers, widths, Wrbf,
           Wj, bj, Wi, bi, Wri1, bri1, Wri2, bri2, Wra1, bra1, Wra2, bra2, u):
    f32 = jnp.float32
    src = edge_index[0].astype(jnp.int32)
    dst = edge_index[1].astype(jnp.int32)
    zpad = jnp.pad(atomic_numbers.astype(jnp.int32), (0, NPAD - N))
    posp = jnp.pad(positions.astype(f32),
                   ((0, NPAD - N), (0, 1))).reshape(NPAD * 4)
    srcp = jnp.pad(src, (0, EPAD - E))
    dstp = jnp.pad(dst, (0, EPAD - E))

    x0, d2 = _sc_prep()(zpad, emb, posp, src, dst)
    d2p = jnp.concatenate(
        [d2, jnp.full((EPAD - E,), 4.0 * RC * RC, f32)]).reshape(NEB, 8, 512)
    gate0, gate1 = _tc_gates(d2p, centers.reshape(K, 1), widths.reshape(K, 1),
                             Wrbf[0], Wrbf[1])
    zer = jnp.zeros((ZROWS, F), f32)

    x = x0
    gates = (gate0, gate1)
    for b in range(2):
        hj, hi = _tc_node(x, Wj[b], bj[b].reshape(1, F), Wi[b],
                          bi[b].reshape(1, F))
        agg = _sc_msg()(gates[b], hj, srcp, dstp, zer).reshape(NC, NPAD, F)
        x = _tc_update(x, hi, agg, Wri1[b], bri1[b].reshape(1, F), Wri2[b],
                       bri2[b].reshape(1, F), Wra1[b], bra1[b].reshape(1, F),
                       Wra2[b], bra2[b].reshape(1, F), u[b].reshape(1, F))
    return x[:N]
